# Initial kernel scaffold; baseline (speedup 1.0000x reference)
import jax
import jax.numpy as jnp
from jax.experimental import pallas as pl


def kernel(idx, table):
    raise NotImplementedError("write your pallas kernel here")


# speedup vs baseline: 4.9461x; 4.9461x over previous
import functools

import jax
import jax.numpy as jnp
from jax import lax
from jax.experimental import pallas as pl
from jax.experimental.pallas import tpu as pltpu
from jax.experimental.pallas import tpu_sc as plsc

NUM_NODES = 1000000
OUT_DIMS = 32
BATCH = 16384
HIST = 200

B_TOTAL = BATCH * HIST
IDX_W = 128
K = 16
CHUNK = K * IDX_W
NC = 2
NS = 16
NW = NC * NS
ROWS_PER_W = B_TOTAL // NW
CHUNKS_PER_W = ROWS_PER_W // CHUNK


@functools.partial(
    pl.kernel,
    mesh=plsc.VectorSubcoreMesh(core_axis_name="c", subcore_axis_name="s"),
    compiler_params=pltpu.CompilerParams(use_tc_tiling_on_sc=False),
    out_type=jax.ShapeDtypeStruct((B_TOTAL, OUT_DIMS), jnp.float32),
    scratch_types=[
        pltpu.VMEM((K, IDX_W), jnp.int32),
        pltpu.VMEM((CHUNK, OUT_DIMS), jnp.float32),
        pltpu.SemaphoreType.DMA,
    ],
)
def _emb_lookup(idx_hbm, table_hbm, out_hbm, idx_v, rows_v, sem):
    wid = lax.axis_index("s") * NC + lax.axis_index("c")
    row0_w = wid * (ROWS_PER_W // IDX_W)

    def body(g, carry):
        row0 = row0_w + g * K
        pltpu.sync_copy(idx_hbm.at[pl.ds(row0, K)], idx_v)
        copies = []
        for j in range(K):
            copies.append(
                pltpu.async_copy(
                    table_hbm.at[idx_v.at[j]],
                    rows_v.at[pl.ds(j * IDX_W, IDX_W)],
                    sem,
                )
            )
        for c in copies:
            c.wait()
        pltpu.sync_copy(rows_v, out_hbm.at[pl.ds(row0 * IDX_W, CHUNK)])
        return carry

    lax.fori_loop(0, CHUNKS_PER_W, body, 0)


def kernel(idx, table):
    b, h = idx.shape
    idx2d = idx.reshape(-1).astype(jnp.int32).reshape(-1, IDX_W)
    out = _emb_lookup(idx2d, table)
    return out.reshape(b, h, table.shape[1])


# speedup vs baseline: 4.9515x; 1.0011x over previous
import functools

import jax
import jax.numpy as jnp
from jax import lax
from jax.experimental import pallas as pl
from jax.experimental.pallas import tpu as pltpu
from jax.experimental.pallas import tpu_sc as plsc

NUM_NODES = 1000000
OUT_DIMS = 32
BATCH = 16384
HIST = 200

B_TOTAL = BATCH * HIST
IDX_W = 128
K = 8
CHUNK = K * IDX_W
NC = 2
NS = 16
NW = NC * NS
ROWS_PER_W = B_TOTAL // NW
CHUNKS_PER_W = ROWS_PER_W // CHUNK
CHUNK_BYTES = CHUNK * OUT_DIMS * 4


@functools.partial(
    pl.kernel,
    mesh=plsc.VectorSubcoreMesh(core_axis_name="c", subcore_axis_name="s"),
    compiler_params=pltpu.CompilerParams(use_tc_tiling_on_sc=False),
    out_type=jax.ShapeDtypeStruct((B_TOTAL, OUT_DIMS), jnp.float32),
    scratch_types=[
        pltpu.VMEM((2, K, IDX_W), jnp.int32),
        pltpu.VMEM((2, CHUNK, OUT_DIMS), jnp.float32),
        pltpu.SemaphoreType.DMA,
        pltpu.SemaphoreType.DMA,
    ],
)
def _emb_lookup(idx_hbm, table_hbm, out_hbm, idx_v, rows_v, gsem, wsem):
    wid = lax.axis_index("s") * NC + lax.axis_index("c")
    row0_w = wid * (ROWS_PER_W // IDX_W)

    def fire(g, slot):
        pltpu.sync_copy(idx_hbm.at[pl.ds(row0_w + g * K, K)], idx_v.at[slot])
        for j in range(K):
            pltpu.async_copy(
                table_hbm.at[idx_v.at[slot, j]],
                rows_v.at[slot, pl.ds(j * IDX_W, IDX_W)],
                gsem,
            )

    def drain_gathers(slot):
        for j in range(K):
            pltpu.make_async_copy(
                table_hbm.at[idx_v.at[slot, j]],
                rows_v.at[slot, pl.ds(j * IDX_W, IDX_W)],
                gsem,
            ).wait()

    def out_slice(g):
        return out_hbm.at[pl.ds((row0_w + g * K) * IDX_W, CHUNK)]

    fire(0, 0)

    def body(g, carry):
        s = g % 2
        ns = 1 - s

        @pl.when(g < CHUNKS_PER_W - 1)
        def _prefetch():
            @pl.when(g >= 1)
            def _reclaim():
                pltpu.make_async_copy(rows_v.at[ns], out_slice(g - 1), wsem).wait()

            fire(g + 1, ns)

        drain_gathers(s)
        pltpu.async_copy(rows_v.at[s], out_slice(g), wsem)
        return carry

    lax.fori_loop(0, CHUNKS_PER_W, body, 0)
    pltpu.make_async_copy(rows_v.at[0], out_slice(0), wsem).wait()
    pltpu.make_async_copy(rows_v.at[0], out_slice(0), wsem).wait()


def kernel(idx, table):
    b, h = idx.shape
    idx2d = idx.reshape(-1).astype(jnp.int32).reshape(-1, IDX_W)
    out = _emb_lookup(idx2d, table)
    return out.reshape(b, h, table.shape[1])
